# baseline (speedup 1.0000x reference)
import functools

import jax
import jax.numpy as jnp
from jax import lax
from jax.experimental import pallas as pl
from jax.experimental.pallas import tpu as pltpu
from jax.experimental.pallas import tpu_sc as plsc

_B = 16384
_F = 39
_NCAT = 26
_NCONT = _F - _NCAT
_V = 100000
_D = 16
_EPS = 1e-5
_OUTW = _NCAT * _D + _NCONT

_NW = 32
_ROWS_PER_W = _B // _NW
_CHUNK = 128
_NCHUNK = _ROWS_PER_W // _CHUNK


def _prelude_body(x_ref, gamma_ref, beta_ref, gidx_ref, cont_ref):
    x = x_ref[...]
    idx = x[:, :_NCAT].astype(jnp.int32)
    offs = lax.broadcasted_iota(jnp.int32, (_B, _NCAT), 1) * _V
    gidx_ref[...] = (idx + offs).T
    xc = x[:, _NCAT:]
    mean = jnp.mean(xc, axis=0, keepdims=True)
    var = jnp.mean((xc - mean) ** 2, axis=0, keepdims=True)
    inv = lax.rsqrt(var + _EPS)
    cont_ref[...] = (xc - mean) * inv * gamma_ref[...] + beta_ref[...]


def _sc_body(tables, gidx, cont, out, idx_v, stage_v, cont_v, sem_g, sem_w):
    wid = lax.axis_index("s") * 2 + lax.axis_index("c")

    def chunk_body(c, carry):
        base = wid * _ROWS_PER_W + c * _CHUNK
        pltpu.sync_copy(gidx.at[:, pl.ds(base, _CHUNK)], idx_v)
        gathers = []
        for f in range(_NCAT):
            gathers.append(
                pltpu.async_copy(tables.at[idx_v.at[f]], stage_v.at[f], sem_g)
            )
        for cp in gathers:
            cp.wait()
        writes = []
        for f in range(_NCAT):
            writes.append(
                pltpu.async_copy(
                    stage_v.at[f],
                    out.at[pl.ds(base, _CHUNK), pl.ds(f * _D, _D)],
                    sem_w,
                )
            )
        pltpu.sync_copy(cont.at[pl.ds(base, _CHUNK), :], cont_v)
        pltpu.sync_copy(cont_v, out.at[pl.ds(base, _CHUNK), pl.ds(_NCAT * _D, _NCONT)])
        for cp in writes:
            cp.wait()
        return carry

    lax.fori_loop(0, _NCHUNK, chunk_body, 0)


@jax.jit
def kernel(x, tables, gamma, beta):
    gidx, cont = pl.pallas_call(
        _prelude_body,
        out_shape=(
            jax.ShapeDtypeStruct((_NCAT, _B), jnp.int32),
            jax.ShapeDtypeStruct((_B, _NCONT), jnp.float32),
        ),
    )(x, gamma.reshape(1, _NCONT), beta.reshape(1, _NCONT))

    sc_call = pl.kernel(
        _sc_body,
        out_type=jax.ShapeDtypeStruct((_B, _OUTW), jnp.float32),
        mesh=plsc.VectorSubcoreMesh(core_axis_name="c", subcore_axis_name="s"),
        scratch_types=[
            pltpu.VMEM((_NCAT, _CHUNK), jnp.int32),
            pltpu.VMEM((_NCAT, _CHUNK, _D), jnp.float32),
            pltpu.VMEM((_CHUNK, _NCONT), jnp.float32),
            pltpu.SemaphoreType.DMA,
            pltpu.SemaphoreType.DMA,
        ],
        compiler_params=pltpu.CompilerParams(use_tc_tiling_on_sc=False),
    )
    return sc_call(tables.reshape(_NCAT * _V, _D), gidx, cont)


# speedup vs baseline: 4.8032x; 4.8032x over previous
import functools

import jax
import jax.numpy as jnp
from jax import lax
from jax.experimental import pallas as pl
from jax.experimental.pallas import tpu as pltpu
from jax.experimental.pallas import tpu_sc as plsc

_B = 16384
_F = 39
_NCAT = 26
_NCONT = _F - _NCAT
_V = 100000
_D = 16
_EPS = 1e-5
_OUTW = _NCAT * _D + _NCONT

_NW = 32
_NTASK = 13
_BCH = 2048
_NBCH = _B // _BCH


def _prelude_body(xT_ref, gamma_ref, beta_ref, contT_ref):
    xc = xT_ref[_NCAT:, :]
    mean = jnp.mean(xc, axis=1, keepdims=True)
    var = jnp.mean((xc - mean) ** 2, axis=1, keepdims=True)
    inv = lax.rsqrt(var + _EPS)
    contT_ref[...] = (xc - mean) * inv * gamma_ref[...] + beta_ref[...]


def _sc_body(xT, tablesT, contT, outT, row_v, fidx_v, outb_v):
    wid = lax.axis_index("s") * 2 + lax.axis_index("c")

    def task_body(k, carry):
        t = k * _NW + wid
        f = t >> 4
        d = t & 15
        pltpu.sync_copy(tablesT.at[f, d, :], row_v)

        def chunk_body(c, carry2):
            b0 = c * _BCH
            pltpu.sync_copy(xT.at[f, pl.ds(b0, _BCH)], fidx_v)

            def g_body(g, carry3):
                idx16 = fidx_v[pl.ds(g * 16, 16)].astype(jnp.int32)
                outb_v[pl.ds(g * 16, 16)] = plsc.load_gather(row_v, [idx16])
                return carry3

            lax.fori_loop(0, _BCH // 16, g_body, 0)
            pltpu.sync_copy(outb_v, outT.at[t, pl.ds(b0, _BCH)])
            return carry2

        lax.fori_loop(0, _NBCH, chunk_body, 0)
        return carry

    lax.fori_loop(0, _NTASK, task_body, 0)

    @pl.when(wid < _NCONT)
    def _():
        pltpu.sync_copy(contT.at[wid, :], row_v.at[pl.ds(0, _B)])
        pltpu.sync_copy(row_v.at[pl.ds(0, _B)], outT.at[_NCAT * _D + wid, :])


@jax.jit
def kernel(x, tables, gamma, beta):
    xT = x.T
    tablesT = jnp.transpose(tables, (0, 2, 1))

    contT = pl.pallas_call(
        _prelude_body,
        out_shape=jax.ShapeDtypeStruct((_NCONT, _B), jnp.float32),
    )(xT, gamma.reshape(_NCONT, 1), beta.reshape(_NCONT, 1))

    sc_call = pl.kernel(
        _sc_body,
        out_type=jax.ShapeDtypeStruct((_OUTW, _B), jnp.float32),
        mesh=plsc.VectorSubcoreMesh(core_axis_name="c", subcore_axis_name="s"),
        scratch_types=[
            pltpu.VMEM((_V,), jnp.float32),
            pltpu.VMEM((_BCH,), jnp.float32),
            pltpu.VMEM((_BCH,), jnp.float32),
        ],
        compiler_params=pltpu.CompilerParams(
            use_tc_tiling_on_sc=True, needs_layout_passes=False
        ),
    )
    outT = sc_call(xT, tablesT, contT)
    return outT.T


# speedup vs baseline: 5.1501x; 1.0722x over previous
import functools

import jax
import jax.numpy as jnp
from jax import lax
from jax.experimental import pallas as pl
from jax.experimental.pallas import tpu as pltpu
from jax.experimental.pallas import tpu_sc as plsc

_B = 16384
_F = 39
_NCAT = 26
_NCONT = _F - _NCAT
_V = 100000
_D = 16
_EPS = 1e-5
_OUTW = _NCAT * _D + _NCONT

_NW = 32
_NTASK = 13
_BCH = 4096
_NBCH = _B // _BCH
_UNROLL = 4


def _prelude_body(xT_ref, gamma_ref, beta_ref, contT_ref):
    xc = xT_ref[_NCAT:, :]
    mean = jnp.mean(xc, axis=1, keepdims=True)
    var = jnp.mean((xc - mean) ** 2, axis=1, keepdims=True)
    inv = lax.rsqrt(var + _EPS)
    contT_ref[...] = (xc - mean) * inv * gamma_ref[...] + beta_ref[...]


def _sc_body(xT, tablesT, contT, outT, row_v, fidx_v, outb_v, sem_r, sem_i, sem_o):
    wid = lax.axis_index("s") * 2 + lax.axis_index("c")

    def task_body(k, carry):
        t = k * _NW + wid
        f = t >> 4
        d = t & 15
        cp_row = pltpu.async_copy(tablesT.at[f, d, :], row_v, sem_r)
        cp_i = pltpu.async_copy(xT.at[f, pl.ds(0, _BCH)], fidx_v.at[0], sem_i)
        cp_row.wait()

        cp_o = [None, None]
        for c in range(_NBCH):
            buf = c % 2
            cp_i.wait()
            if c + 1 < _NBCH:
                cp_i = pltpu.async_copy(
                    xT.at[f, pl.ds((c + 1) * _BCH, _BCH)],
                    fidx_v.at[1 - buf],
                    sem_i,
                )
            if cp_o[buf] is not None:
                cp_o[buf].wait()

            def g_body(g, carry3):
                for j in range(_UNROLL):
                    o = (g * _UNROLL + j) * 16
                    idx16 = fidx_v[buf, pl.ds(o, 16)].astype(jnp.int32)
                    outb_v[buf, pl.ds(o, 16)] = plsc.load_gather(row_v, [idx16])
                return carry3

            lax.fori_loop(0, _BCH // 16 // _UNROLL, g_body, 0)
            cp_o[buf] = pltpu.async_copy(
                outb_v.at[buf], outT.at[t, pl.ds(c * _BCH, _BCH)], sem_o
            )
        for cp in cp_o:
            cp.wait()
        return carry

    lax.fori_loop(0, _NTASK, task_body, 0)

    @pl.when(wid < _NCONT)
    def _():
        pltpu.sync_copy(contT.at[wid, :], row_v.at[pl.ds(0, _B)])
        pltpu.sync_copy(row_v.at[pl.ds(0, _B)], outT.at[_NCAT * _D + wid, :])


@jax.jit
def kernel(x, tables, gamma, beta):
    xT = x.T
    tablesT = jnp.transpose(tables, (0, 2, 1))

    contT = pl.pallas_call(
        _prelude_body,
        out_shape=jax.ShapeDtypeStruct((_NCONT, _B), jnp.float32),
    )(xT, gamma.reshape(_NCONT, 1), beta.reshape(_NCONT, 1))

    sc_call = pl.kernel(
        _sc_body,
        out_type=jax.ShapeDtypeStruct((_OUTW, _B), jnp.float32),
        mesh=plsc.VectorSubcoreMesh(core_axis_name="c", subcore_axis_name="s"),
        scratch_types=[
            pltpu.VMEM((_V,), jnp.float32),
            pltpu.VMEM((2, _BCH), jnp.float32),
            pltpu.VMEM((2, _BCH), jnp.float32),
            pltpu.SemaphoreType.DMA,
            pltpu.SemaphoreType.DMA,
            pltpu.SemaphoreType.DMA,
        ],
        compiler_params=pltpu.CompilerParams(
            use_tc_tiling_on_sc=True, needs_layout_passes=False
        ),
    )
    outT = sc_call(xT, tablesT, contT)
    return outT.T
